# baseline (speedup 1.0000x reference)
import functools

import jax
import jax.numpy as jnp
from jax import lax
from jax.experimental import pallas as pl
from jax.experimental.pallas import tpu as pltpu
from jax.experimental.pallas import tpu_sc as plsc

NEW_TOKEN_ID = 1000000
LANES = 16
SUB = 128
CHUNK = 512
NSUB = CHUNK // SUB


@functools.lru_cache(maxsize=None)
def _build_lookup(B, V, D):
    mesh = plsc.VectorSubcoreMesh(core_axis_name="c", subcore_axis_name="s")
    NC, NS = mesh.num_cores, mesh.num_subcores
    NW = NC * NS
    assert B % NW == 0
    b_per_w = B // NW
    assert b_per_w % CHUNK == 0
    G = b_per_w // CHUNK
    assert G % 2 == 0 and G >= 2
    GG = G // 2
    assert D == 4 * LANES

    @functools.partial(
        pl.kernel,
        out_type=jax.ShapeDtypeStruct((B, D), jnp.float32),
        mesh=mesh,
        compiler_params=pltpu.CompilerParams(use_tc_tiling_on_sc=False,
                                             needs_layout_passes=False),
        scratch_types=[
            pltpu.VMEM((CHUNK,), jnp.int32),
            pltpu.VMEM((CHUNK,), jnp.int32),
            pltpu.VMEM((CHUNK, D), jnp.float32),
            pltpu.VMEM((CHUNK,), jnp.int32),
            pltpu.VMEM((CHUNK,), jnp.int32),
            pltpu.VMEM((CHUNK, D), jnp.float32),
            pltpu.VMEM((D,), jnp.float32),
            pltpu.SemaphoreType.DMA,
            pltpu.SemaphoreType.DMA,
        ],
    )
    def lookup(tbl, ids, nw, out, idxr_a, idxs_a, rows_a, idxr_b, idxs_b,
               rows_b, nw_v, sem_a, sem_b):
        wid = lax.axis_index("s") * NC + lax.axis_index("c")
        wbase = wid * b_per_w
        pltpu.sync_copy(nw, nw_v)

        def fire(g, idxr, idxs, rows, sem):
            base = wbase + g * CHUNK
            pltpu.sync_copy(ids.at[pl.ds(base, CHUNK)], idxr)

            def p1(i, acc):
                v = idxr[pl.ds(i * LANES, LANES)]
                m = v == NEW_TOKEN_ID
                idxs[pl.ds(i * LANES, LANES)] = jnp.where(m, 0, v)
                return acc | jnp.any(m)

            acc = lax.fori_loop(0, CHUNK // LANES, p1, jnp.bool_(False))
            for j in range(NSUB):
                pltpu.async_copy(tbl.at[idxs.at[pl.ds(j * SUB, SUB)]],
                                 rows.at[pl.ds(j * SUB, SUB)], sem)
            return acc

        def finish(g, flag, idxr, idxs, rows, sem):
            for j in range(NSUB):
                pltpu.make_async_copy(tbl.at[idxs.at[pl.ds(j * SUB, SUB)]],
                                      rows.at[pl.ds(j * SUB, SUB)], sem).wait()

            @pl.when(flag)
            def _():
                def grp(i, carry):
                    v = idxr[pl.ds(i * LANES, LANES)]
                    m = v == NEW_TOKEN_ID

                    @pl.when(jnp.any(m))
                    def __():
                        ri = i * LANES + lax.iota(jnp.int32, LANES)
                        for c in range(D):
                            ci = jnp.full((LANES,), c, jnp.int32)
                            xv = plsc.load_gather(nw_v, [ci])
                            plsc.store_scatter(rows, [ri, ci], xv, mask=m)

                    return carry

                lax.fori_loop(0, CHUNK // LANES, grp, 0)

            pltpu.sync_copy(rows, out.at[pl.ds(wbase + g * CHUNK, CHUNK)])

        f_a = fire(0, idxr_a, idxs_a, rows_a, sem_a)

        def body(i, f_a):
            g0 = 2 * i
            f_b = fire(g0 + 1, idxr_b, idxs_b, rows_b, sem_b)
            finish(g0, f_a, idxr_a, idxs_a, rows_a, sem_a)
            f_a2 = fire(g0 + 2, idxr_a, idxs_a, rows_a, sem_a)
            finish(g0 + 1, f_b, idxr_b, idxs_b, rows_b, sem_b)
            return f_a2

        f_a = lax.fori_loop(0, GG - 1, body, f_a)
        f_b = fire(G - 1, idxr_b, idxs_b, rows_b, sem_b)
        finish(G - 2, f_a, idxr_a, idxs_a, rows_a, sem_a)
        finish(G - 1, f_b, idxr_b, idxs_b, rows_b, sem_b)

    return lookup


def kernel(input_ids, base_weight, new_weight):
    batch, seq = input_ids.shape
    V, D = base_weight.shape
    ids = input_ids.reshape(-1).astype(jnp.int32)
    nw = new_weight.reshape(-1)
    lookup = _build_lookup(ids.shape[0], V, D)
    out = lookup(base_weight, ids, nw)
    return out.reshape(batch, seq, D)


# speedup vs baseline: 1.0247x; 1.0247x over previous
import functools

import jax
import jax.numpy as jnp
from jax import lax
from jax.experimental import pallas as pl
from jax.experimental.pallas import tpu as pltpu
from jax.experimental.pallas import tpu_sc as plsc

NEW_TOKEN_ID = 1000000
LANES = 16
SUB = 128
CHUNK = 512
NSUB = CHUNK // SUB


@functools.lru_cache(maxsize=None)
def _build_lookup(B, V, D):
    mesh = plsc.VectorSubcoreMesh(core_axis_name="c", subcore_axis_name="s")
    NC, NS = mesh.num_cores, mesh.num_subcores
    NW = NC * NS
    assert B % NW == 0
    b_per_w = B // NW
    assert b_per_w % CHUNK == 0
    G = b_per_w // CHUNK
    assert G % 2 == 0 and G >= 2
    GG = G // 2
    assert D == 4 * LANES

    @functools.partial(
        pl.kernel,
        out_type=jax.ShapeDtypeStruct((B, D), jnp.float32),
        mesh=mesh,
        compiler_params=pltpu.CompilerParams(use_tc_tiling_on_sc=False,
                                             needs_layout_passes=False),
        scratch_types=[
            pltpu.VMEM((CHUNK,), jnp.int32),
            pltpu.VMEM((CHUNK,), jnp.int32),
            pltpu.VMEM((CHUNK, D), jnp.float32),
            pltpu.VMEM((CHUNK,), jnp.int32),
            pltpu.VMEM((CHUNK,), jnp.int32),
            pltpu.VMEM((CHUNK, D), jnp.float32),
            pltpu.VMEM((D,), jnp.float32),
            pltpu.SemaphoreType.DMA,
            pltpu.SemaphoreType.DMA,
        ],
    )
    def lookup(tbl, ids, nw, out, idxr_a, idxs_a, rows_a, idxr_b, idxs_b,
               rows_b, nw_v, sem_a, sem_b):
        wid = lax.axis_index("s") * NC + lax.axis_index("c")
        wbase = wid * b_per_w
        pltpu.sync_copy(nw, nw_v)

        def fire(g, idxr, idxs, rows, sem):
            base = wbase + g * CHUNK
            pltpu.sync_copy(ids.at[pl.ds(base, CHUNK)], idxr)

            def p1(i, acc):
                v = idxr[pl.ds(i * LANES, LANES)]
                m = v == NEW_TOKEN_ID
                idxs[pl.ds(i * LANES, LANES)] = jnp.where(m, 0, v)
                return acc | jnp.any(m)

            acc = lax.fori_loop(0, CHUNK // LANES, p1, jnp.bool_(False))
            for j in range(NSUB):
                pltpu.async_copy(tbl.at[idxs.at[pl.ds(j * SUB, SUB)]],
                                 rows.at[pl.ds(j * SUB, SUB)], sem)
            return acc

        def finish(g, flag, idxr, idxs, rows, sem):
            for j in range(NSUB):
                pltpu.make_async_copy(tbl.at[idxs.at[pl.ds(j * SUB, SUB)]],
                                      rows.at[pl.ds(j * SUB, SUB)], sem).wait()

            @pl.when(flag)
            def _():
                def grp(i, carry):
                    v = idxr[pl.ds(i * LANES, LANES)]
                    m = v == NEW_TOKEN_ID

                    @pl.when(jnp.any(m))
                    def __():
                        ri = i * LANES + lax.iota(jnp.int32, LANES)
                        for c in range(D):
                            ci = jnp.full((LANES,), c, jnp.int32)
                            xv = plsc.load_gather(nw_v, [ci])
                            plsc.store_scatter(rows, [ri, ci], xv, mask=m)

                    return carry

                lax.fori_loop(0, CHUNK // LANES, grp, 0)

            pltpu.sync_copy(rows, out.at[pl.ds(wbase + g * CHUNK, CHUNK)])

        f_a = fire(0, idxr_a, idxs_a, rows_a, sem_a)

        def body(i, f_a):
            g0 = 2 * i
            f_b = fire(g0 + 1, idxr_b, idxs_b, rows_b, sem_b)
            finish(g0, f_a, idxr_a, idxs_a, rows_a, sem_a)
            f_a2 = fire(g0 + 2, idxr_a, idxs_a, rows_a, sem_a)
            finish(g0 + 1, f_b, idxr_b, idxs_b, rows_b, sem_b)
            return f_a2

        f_a = lax.fori_loop(0, GG - 1, body, f_a)
        f_b = fire(G - 1, idxr_b, idxs_b, rows_b, sem_b)
        finish(G - 2, f_a, idxr_a, idxs_a, rows_a, sem_a)
        finish(G - 1, f_b, idxr_b, idxs_b, rows_b, sem_b)

    return lookup


def kernel(input_ids, base_weight, new_weight):
    batch, seq = input_ids.shape
    V, D = base_weight.shape
    ids = input_ids.T.reshape(-1).astype(jnp.int32)
    nw = new_weight.reshape(-1)
    lookup = _build_lookup(ids.shape[0], V, D)
    out = lookup(base_weight, ids, nw)
    return out.reshape(seq, batch, D).swapaxes(0, 1)
